# Initial kernel scaffold; baseline (speedup 1.0000x reference)
import functools

import jax
import jax.numpy as jnp
from jax import lax
from jax.experimental import pallas as pl
from jax.experimental.pallas import tpu as pltpu
from jax.experimental.pallas import tpu_sc as plsc

_N = 10000
_E = 320000
_DF = 128
_DH = 16
_DC = 40

_NC = 2
_NS = 16
_NW = _NC * _NS
_CH = 128
_NCH = 80
_EPAD = _NW * _NCH * _CH
_NPAD = 10240
_RPS = _NPAD // _NS
_RBLK = 1024

_MESH = plsc.VectorSubcoreMesh(
    core_axis_name="c", subcore_axis_name="s", num_cores=_NC, num_subcores=_NS
)


def _deg_body(dst_hbm, zero_hbm, out_hbm, dstv, onesv, accum):
    c = lax.axis_index("c")
    s = lax.axis_index("s")
    wid = s * _NC + c
    rows = pl.ds(s * _RPS, _RPS)
    pltpu.sync_copy(zero_hbm.at[rows], accum.at[rows])
    pltpu.sync_copy(dst_hbm.at[wid], dstv)
    for i in range(_CH // 16):
        onesv[pl.ds(i * 16, 16)] = jnp.ones((16,), jnp.float32)
    plsc.subcore_barrier()

    def step(j, carry):
        pltpu.sync_copy(onesv, accum.at[dstv.at[j]], add=True)
        return carry

    lax.fori_loop(0, _NCH, step, 0)
    plsc.subcore_barrier()
    pltpu.sync_copy(accum.at[rows], out_hbm.at[c, rows])


_deg = functools.partial(
    pl.kernel,
    out_type=jax.ShapeDtypeStruct((_NC, _NPAD), jnp.float32),
    mesh=_MESH,
    scratch_types=[
        pltpu.VMEM((_NCH, _CH), jnp.int32),
        pltpu.VMEM((_CH,), jnp.float32),
        pltpu.VMEM_SHARED((_NPAD,), jnp.float32),
    ],
)(_deg_body)


def _agg_body(h_hbm, src_hbm, dst_hbm, zero_hbm, out_hbm,
              srcv, dstv, msga, msgb, accum, gsa, gsb):
    c = lax.axis_index("c")
    s = lax.axis_index("s")
    wid = s * _NC + c
    rows = pl.ds(s * _RPS, _RPS)

    @pl.when(c == 0)
    def _():
        pltpu.sync_copy(h_hbm.at[rows], accum.at[rows])

    @pl.when(c != 0)
    def _():
        pltpu.sync_copy(zero_hbm.at[rows], accum.at[rows])

    pltpu.sync_copy(src_hbm.at[wid], srcv)
    pltpu.sync_copy(dst_hbm.at[wid], dstv)
    plsc.subcore_barrier()

    def step(j, carry):
        ga = pltpu.async_copy(h_hbm.at[srcv.at[2 * j]], msga, gsa)
        gb = pltpu.async_copy(h_hbm.at[srcv.at[2 * j + 1]], msgb, gsb)
        ga.wait()
        pltpu.sync_copy(msga, accum.at[dstv.at[2 * j]], add=True)
        gb.wait()
        pltpu.sync_copy(msgb, accum.at[dstv.at[2 * j + 1]], add=True)
        return carry

    lax.fori_loop(0, _NCH // 2, step, 0)
    plsc.subcore_barrier()
    pltpu.sync_copy(accum.at[rows], out_hbm.at[c, rows])


_agg = functools.partial(
    pl.kernel,
    out_type=jax.ShapeDtypeStruct((_NC, _NPAD, _DH), jnp.float32),
    mesh=_MESH,
    scratch_types=[
        pltpu.VMEM((_NCH, _CH), jnp.int32),
        pltpu.VMEM((_NCH, _CH), jnp.int32),
        pltpu.VMEM((_CH, _DH), jnp.float32),
        pltpu.VMEM((_CH, _DH), jnp.float32),
        pltpu.VMEM_SHARED((_NPAD, _DH), jnp.float32),
        pltpu.SemaphoreType.DMA,
        pltpu.SemaphoreType.DMA,
    ],
)(_agg_body)


def _tc1_body(x_ref, w1_ref, p0_ref, p1_ref, h_ref):
    dis = lax.rsqrt(p0_ref[...] + p1_ref[...] + 1.0)
    h = jnp.dot(x_ref[...], w1_ref[...], preferred_element_type=jnp.float32)
    h_ref[...] = h * dis


_tc1 = pl.pallas_call(
    _tc1_body,
    grid=(_NPAD // _RBLK,),
    in_specs=[
        pl.BlockSpec((_RBLK, _DF), lambda i: (i, 0)),
        pl.BlockSpec((_DF, _DH), lambda i: (0, 0)),
        pl.BlockSpec((_RBLK, 1), lambda i: (i, 0)),
        pl.BlockSpec((_RBLK, 1), lambda i: (i, 0)),
    ],
    out_specs=pl.BlockSpec((_RBLK, _DH), lambda i: (i, 0)),
    out_shape=jax.ShapeDtypeStruct((_NPAD, _DH), jnp.float32),
)


def _tc2_body(a0_ref, a1_ref, p0_ref, p1_ref, b1_ref, z_ref):
    dis = lax.rsqrt(p0_ref[...] + p1_ref[...] + 1.0)
    agg = dis * (a0_ref[...] + a1_ref[...]) + b1_ref[...]
    z_ref[...] = dis * jnp.maximum(agg, 0.0)


_tc2 = pl.pallas_call(
    _tc2_body,
    grid=(_NPAD // _RBLK,),
    in_specs=[
        pl.BlockSpec((_RBLK, _DH), lambda i: (i, 0)),
        pl.BlockSpec((_RBLK, _DH), lambda i: (i, 0)),
        pl.BlockSpec((_RBLK, 1), lambda i: (i, 0)),
        pl.BlockSpec((_RBLK, 1), lambda i: (i, 0)),
        pl.BlockSpec((1, _DH), lambda i: (0, 0)),
    ],
    out_specs=pl.BlockSpec((_RBLK, _DH), lambda i: (i, 0)),
    out_shape=jax.ShapeDtypeStruct((_NPAD, _DH), jnp.float32),
)


def _tc3_body(a0_ref, a1_ref, p0_ref, p1_ref, w2_ref, b2_ref, o_ref):
    dis = lax.rsqrt(p0_ref[...] + p1_ref[...] + 1.0)
    agg = dis * (a0_ref[...] + a1_ref[...])
    logits = jnp.dot(agg, w2_ref[...], preferred_element_type=jnp.float32)
    logits = logits + b2_ref[...]
    m = jnp.max(logits, axis=1, keepdims=True)
    lse = jnp.log(jnp.sum(jnp.exp(logits - m), axis=1, keepdims=True)) + m
    o_ref[...] = logits - lse


_tc3 = pl.pallas_call(
    _tc3_body,
    grid=(_NPAD // _RBLK,),
    in_specs=[
        pl.BlockSpec((_RBLK, _DH), lambda i: (i, 0)),
        pl.BlockSpec((_RBLK, _DH), lambda i: (i, 0)),
        pl.BlockSpec((_RBLK, 1), lambda i: (i, 0)),
        pl.BlockSpec((_RBLK, 1), lambda i: (i, 0)),
        pl.BlockSpec((_DH, _DC), lambda i: (0, 0)),
        pl.BlockSpec((1, _DC), lambda i: (0, 0)),
    ],
    out_specs=pl.BlockSpec((_RBLK, _DC), lambda i: (i, 0)),
    out_shape=jax.ShapeDtypeStruct((_NPAD, _DC), jnp.float32),
)


def kernel(x, edge_index, W1, b1, W2, b2):
    src = edge_index[0].astype(jnp.int32)
    dst = edge_index[1].astype(jnp.int32)
    pad_e = _EPAD - _E
    src_p = jnp.concatenate([src, jnp.zeros((pad_e,), jnp.int32)])
    dst_p = jnp.concatenate([dst, jnp.full((pad_e,), _N, jnp.int32)])
    src_p = src_p.reshape(_NW, _NCH, _CH)
    dst_p = dst_p.reshape(_NW, _NCH, _CH)
    zeros1 = jnp.zeros((_NPAD,), jnp.float32)
    zeros2 = jnp.zeros((_NPAD, _DH), jnp.float32)
    x_p = jnp.pad(x, ((0, _NPAD - _N), (0, 0)))

    degp = _deg(dst_p, zeros1)
    p0 = degp[0][:, None]
    p1 = degp[1][:, None]
    h = _tc1(x_p, W1, p0, p1)
    a = _agg(h, src_p, dst_p, zeros2)
    z = _tc2(a[0], a[1], p0, p1, b1.reshape(1, _DH))
    a2 = _agg(z, src_p, dst_p, zeros2)
    out = _tc3(a2[0], a2[1], p0, p1, W2, b2.reshape(1, _DC))
    return out[:_N]


# speedup vs baseline: 32.4565x; 32.4565x over previous
import functools

import jax
import jax.numpy as jnp
from jax import lax
from jax.experimental import pallas as pl
from jax.experimental.pallas import tpu as pltpu
from jax.experimental.pallas import tpu_sc as plsc

_N = 10000
_E = 320000
_DF = 128
_DH = 16
_DC = 40

_NC = 2
_NS = 16
_NW = _NC * _NS
_CH = 128
_NCH = 80
_EPAD = _NW * _NCH * _CH
_NPAD = 10240
_RPS = _NPAD // _NS
_RBLK = 1024

_MESH = plsc.VectorSubcoreMesh(
    core_axis_name="c", subcore_axis_name="s", num_cores=_NC, num_subcores=_NS
)


def _deg_body(dst_hbm, zero_hbm, out_hbm, dstv, onesv, accum):
    c = lax.axis_index("c")
    s = lax.axis_index("s")
    wid = s * _NC + c
    rows = pl.ds(s * _RPS, _RPS)
    pltpu.sync_copy(zero_hbm.at[rows], accum.at[rows])
    pltpu.sync_copy(dst_hbm.at[wid], dstv)
    for i in range(_CH // 16):
        onesv[pl.ds(i * 16, 16)] = jnp.ones((16,), jnp.float32)
    plsc.subcore_barrier()

    def step(j, carry):
        pltpu.sync_copy(onesv, accum.at[dstv.at[j]], add=True)
        return carry

    lax.fori_loop(0, _NCH, step, 0)
    plsc.subcore_barrier()
    pltpu.sync_copy(accum.at[rows], out_hbm.at[c, rows])


_deg = functools.partial(
    pl.kernel,
    out_type=jax.ShapeDtypeStruct((_NC, _NPAD), jnp.float32),
    mesh=_MESH,
    scratch_types=[
        pltpu.VMEM((_NCH, _CH), jnp.int32),
        pltpu.VMEM((_CH,), jnp.float32),
        pltpu.VMEM_SHARED((_NPAD,), jnp.float32),
    ],
    compiler_params=pltpu.CompilerParams(use_tc_tiling_on_sc=False),
)(_deg_body)


def _agg_body(h_hbm, src_hbm, dst_hbm, zero_hbm, out_hbm,
              srcv, dstv, msga, msgb, accum, gsa, gsb):
    c = lax.axis_index("c")
    s = lax.axis_index("s")
    wid = s * _NC + c
    rows = pl.ds(s * _RPS, _RPS)

    @pl.when(c == 0)
    def _():
        pltpu.sync_copy(h_hbm.at[rows], accum.at[rows])

    @pl.when(c != 0)
    def _():
        pltpu.sync_copy(zero_hbm.at[rows], accum.at[rows])

    pltpu.sync_copy(src_hbm.at[wid], srcv)
    pltpu.sync_copy(dst_hbm.at[wid], dstv)
    plsc.subcore_barrier()

    def step(j, carry):
        ga = pltpu.async_copy(h_hbm.at[srcv.at[2 * j]], msga, gsa)
        gb = pltpu.async_copy(h_hbm.at[srcv.at[2 * j + 1]], msgb, gsb)
        ga.wait()
        pltpu.sync_copy(msga, accum.at[dstv.at[2 * j]], add=True)
        gb.wait()
        pltpu.sync_copy(msgb, accum.at[dstv.at[2 * j + 1]], add=True)
        return carry

    lax.fori_loop(0, _NCH // 2, step, 0)
    plsc.subcore_barrier()
    pltpu.sync_copy(accum.at[rows], out_hbm.at[c, rows])


_agg = functools.partial(
    pl.kernel,
    out_type=jax.ShapeDtypeStruct((_NC, _NPAD, _DH), jnp.float32),
    mesh=_MESH,
    scratch_types=[
        pltpu.VMEM((_NCH, _CH), jnp.int32),
        pltpu.VMEM((_NCH, _CH), jnp.int32),
        pltpu.VMEM((_CH, _DH), jnp.float32),
        pltpu.VMEM((_CH, _DH), jnp.float32),
        pltpu.VMEM_SHARED((_NPAD, _DH), jnp.float32),
        pltpu.SemaphoreType.DMA,
        pltpu.SemaphoreType.DMA,
    ],
    compiler_params=pltpu.CompilerParams(use_tc_tiling_on_sc=False),
)(_agg_body)


def _tc1_body(x_ref, w1_ref, p0_ref, p1_ref, h_ref):
    dis = lax.rsqrt(p0_ref[...] + p1_ref[...] + 1.0)
    h = jnp.dot(x_ref[...], w1_ref[...], preferred_element_type=jnp.float32)
    h_ref[...] = h * dis


_tc1 = pl.pallas_call(
    _tc1_body,
    grid=(_NPAD // _RBLK,),
    in_specs=[
        pl.BlockSpec((_RBLK, _DF), lambda i: (i, 0)),
        pl.BlockSpec((_DF, _DH), lambda i: (0, 0)),
        pl.BlockSpec((_RBLK, 1), lambda i: (i, 0)),
        pl.BlockSpec((_RBLK, 1), lambda i: (i, 0)),
    ],
    out_specs=pl.BlockSpec((_RBLK, _DH), lambda i: (i, 0)),
    out_shape=jax.ShapeDtypeStruct((_NPAD, _DH), jnp.float32),
)


def _tc2_body(a0_ref, a1_ref, p0_ref, p1_ref, b1_ref, z_ref):
    dis = lax.rsqrt(p0_ref[...] + p1_ref[...] + 1.0)
    agg = dis * (a0_ref[...] + a1_ref[...]) + b1_ref[...]
    z_ref[...] = dis * jnp.maximum(agg, 0.0)


_tc2 = pl.pallas_call(
    _tc2_body,
    grid=(_NPAD // _RBLK,),
    in_specs=[
        pl.BlockSpec((_RBLK, _DH), lambda i: (i, 0)),
        pl.BlockSpec((_RBLK, _DH), lambda i: (i, 0)),
        pl.BlockSpec((_RBLK, 1), lambda i: (i, 0)),
        pl.BlockSpec((_RBLK, 1), lambda i: (i, 0)),
        pl.BlockSpec((1, _DH), lambda i: (0, 0)),
    ],
    out_specs=pl.BlockSpec((_RBLK, _DH), lambda i: (i, 0)),
    out_shape=jax.ShapeDtypeStruct((_NPAD, _DH), jnp.float32),
)


def _tc3_body(a0_ref, a1_ref, p0_ref, p1_ref, w2_ref, b2_ref, o_ref):
    dis = lax.rsqrt(p0_ref[...] + p1_ref[...] + 1.0)
    agg = dis * (a0_ref[...] + a1_ref[...])
    logits = jnp.dot(agg, w2_ref[...], preferred_element_type=jnp.float32)
    logits = logits + b2_ref[...]
    m = jnp.max(logits, axis=1, keepdims=True)
    lse = jnp.log(jnp.sum(jnp.exp(logits - m), axis=1, keepdims=True)) + m
    o_ref[...] = logits - lse


_tc3 = pl.pallas_call(
    _tc3_body,
    grid=(_NPAD // _RBLK,),
    in_specs=[
        pl.BlockSpec((_RBLK, _DH), lambda i: (i, 0)),
        pl.BlockSpec((_RBLK, _DH), lambda i: (i, 0)),
        pl.BlockSpec((_RBLK, 1), lambda i: (i, 0)),
        pl.BlockSpec((_RBLK, 1), lambda i: (i, 0)),
        pl.BlockSpec((_DH, _DC), lambda i: (0, 0)),
        pl.BlockSpec((1, _DC), lambda i: (0, 0)),
    ],
    out_specs=pl.BlockSpec((_RBLK, _DC), lambda i: (i, 0)),
    out_shape=jax.ShapeDtypeStruct((_NPAD, _DC), jnp.float32),
)


def kernel(x, edge_index, W1, b1, W2, b2):
    src = edge_index[0].astype(jnp.int32)
    dst = edge_index[1].astype(jnp.int32)
    pad_e = _EPAD - _E
    src_p = jnp.concatenate([src, jnp.zeros((pad_e,), jnp.int32)])
    dst_p = jnp.concatenate([dst, jnp.full((pad_e,), _N, jnp.int32)])
    src_p = src_p.reshape(_NW, _NCH, _CH)
    dst_p = dst_p.reshape(_NW, _NCH, _CH)
    zeros1 = jnp.zeros((_NPAD,), jnp.float32)
    zeros2 = jnp.zeros((_NPAD, _DH), jnp.float32)
    x_p = jnp.pad(x, ((0, _NPAD - _N), (0, 0)))

    degp = _deg(dst_p, zeros1)
    p0 = degp[0][:, None]
    p1 = degp[1][:, None]
    h = _tc1(x_p, W1, p0, p1)
    a = _agg(h, src_p, dst_p, zeros2)
    z = _tc2(a[0], a[1], p0, p1, b1.reshape(1, _DH))
    a2 = _agg(z, src_p, dst_p, zeros2)
    out = _tc3(a2[0], a2[1], p0, p1, W2, b2.reshape(1, _DC))
    return out[:_N]


# speedup vs baseline: 36.5039x; 1.1247x over previous
import functools

import jax
import jax.numpy as jnp
from jax import lax
from jax.experimental import pallas as pl
from jax.experimental.pallas import tpu as pltpu
from jax.experimental.pallas import tpu_sc as plsc

_N = 10000
_E = 320000
_DF = 128
_DH = 16
_DC = 40

_NC = 2
_NS = 16
_NW = _NC * _NS
_CH = 128
_NCH = 80
_EPAD = _NW * _NCH * _CH
_NPAD = 10240
_RPS = _NPAD // _NS
_RBLK = 1024

_MESH = plsc.VectorSubcoreMesh(
    core_axis_name="c", subcore_axis_name="s", num_cores=_NC, num_subcores=_NS
)


def _deg_body(dst_hbm, zero_hbm, out_hbm, dstv, onesv, accum, dsem):
    c = lax.axis_index("c")
    s = lax.axis_index("s")
    wid = s * _NC + c
    rows = pl.ds(s * _RPS, _RPS)
    pltpu.sync_copy(zero_hbm.at[rows], accum.at[rows])
    pltpu.sync_copy(dst_hbm.at[wid], dstv)
    for i in range(_CH // 16):
        onesv[pl.ds(i * 16, 16)] = jnp.ones((16,), jnp.float32)
    plsc.subcore_barrier()

    def step(g, carry):
        descs = [
            pltpu.async_copy(onesv, accum.at[dstv.at[8 * g + b]], dsem.at[b], add=True)
            for b in range(8)
        ]
        for d in descs:
            d.wait()
        return carry

    lax.fori_loop(0, _NCH // 8, step, 0)
    plsc.subcore_barrier()
    pltpu.sync_copy(accum.at[rows], out_hbm.at[c, rows])


_deg = functools.partial(
    pl.kernel,
    out_type=jax.ShapeDtypeStruct((_NC, _NPAD), jnp.float32),
    mesh=_MESH,
    scratch_types=[
        pltpu.VMEM((_NCH, _CH), jnp.int32),
        pltpu.VMEM((_CH,), jnp.float32),
        pltpu.VMEM_SHARED((_NPAD,), jnp.float32),
        pltpu.SemaphoreType.DMA((8,)),
    ],
    compiler_params=pltpu.CompilerParams(use_tc_tiling_on_sc=False),
)(_deg_body)


def _agg_body(h_hbm, src_hbm, dst_hbm, zero_hbm, out_hbm,
              srcv, dstv, msg, accum, gsem, ssem):
    c = lax.axis_index("c")
    s = lax.axis_index("s")
    wid = s * _NC + c
    rows = pl.ds(s * _RPS, _RPS)

    @pl.when(c == 0)
    def _():
        pltpu.sync_copy(h_hbm.at[rows], accum.at[rows])

    @pl.when(c != 0)
    def _():
        pltpu.sync_copy(zero_hbm.at[rows], accum.at[rows])

    pltpu.sync_copy(src_hbm.at[wid], srcv)
    pltpu.sync_copy(dst_hbm.at[wid], dstv)
    plsc.subcore_barrier()

    def step(g, carry):
        base = 8 * g
        gd = [
            pltpu.async_copy(h_hbm.at[srcv.at[base + b]], msg.at[b], gsem.at[b])
            for b in range(8)
        ]
        sd = []
        for b in range(8):
            gd[b].wait()
            sd.append(pltpu.async_copy(
                msg.at[b], accum.at[dstv.at[base + b]], ssem.at[b], add=True))
        for d in sd:
            d.wait()
        return carry

    lax.fori_loop(0, _NCH // 8, step, 0)
    plsc.subcore_barrier()
    pltpu.sync_copy(accum.at[rows], out_hbm.at[c, rows])


_agg = functools.partial(
    pl.kernel,
    out_type=jax.ShapeDtypeStruct((_NC, _NPAD, _DH), jnp.float32),
    mesh=_MESH,
    scratch_types=[
        pltpu.VMEM((_NCH, _CH), jnp.int32),
        pltpu.VMEM((_NCH, _CH), jnp.int32),
        pltpu.VMEM((8, _CH, _DH), jnp.float32),
        pltpu.VMEM_SHARED((_NPAD, _DH), jnp.float32),
        pltpu.SemaphoreType.DMA((8,)),
        pltpu.SemaphoreType.DMA((8,)),
    ],
    compiler_params=pltpu.CompilerParams(use_tc_tiling_on_sc=False),
)(_agg_body)


def _tc1_body(x_ref, w1_ref, p0_ref, p1_ref, h_ref):
    dis = lax.rsqrt(p0_ref[...] + p1_ref[...] + 1.0)
    h = jnp.dot(x_ref[...], w1_ref[...], preferred_element_type=jnp.float32)
    h_ref[...] = h * dis


_tc1 = pl.pallas_call(
    _tc1_body,
    grid=(_NPAD // _RBLK,),
    in_specs=[
        pl.BlockSpec((_RBLK, _DF), lambda i: (i, 0)),
        pl.BlockSpec((_DF, _DH), lambda i: (0, 0)),
        pl.BlockSpec((_RBLK, 1), lambda i: (i, 0)),
        pl.BlockSpec((_RBLK, 1), lambda i: (i, 0)),
    ],
    out_specs=pl.BlockSpec((_RBLK, _DH), lambda i: (i, 0)),
    out_shape=jax.ShapeDtypeStruct((_NPAD, _DH), jnp.float32),
)


def _tc2_body(a0_ref, a1_ref, p0_ref, p1_ref, b1_ref, z_ref):
    dis = lax.rsqrt(p0_ref[...] + p1_ref[...] + 1.0)
    agg = dis * (a0_ref[...] + a1_ref[...]) + b1_ref[...]
    z_ref[...] = dis * jnp.maximum(agg, 0.0)


_tc2 = pl.pallas_call(
    _tc2_body,
    grid=(_NPAD // _RBLK,),
    in_specs=[
        pl.BlockSpec((_RBLK, _DH), lambda i: (i, 0)),
        pl.BlockSpec((_RBLK, _DH), lambda i: (i, 0)),
        pl.BlockSpec((_RBLK, 1), lambda i: (i, 0)),
        pl.BlockSpec((_RBLK, 1), lambda i: (i, 0)),
        pl.BlockSpec((1, _DH), lambda i: (0, 0)),
    ],
    out_specs=pl.BlockSpec((_RBLK, _DH), lambda i: (i, 0)),
    out_shape=jax.ShapeDtypeStruct((_NPAD, _DH), jnp.float32),
)


def _tc3_body(a0_ref, a1_ref, p0_ref, p1_ref, w2_ref, b2_ref, o_ref):
    dis = lax.rsqrt(p0_ref[...] + p1_ref[...] + 1.0)
    agg = dis * (a0_ref[...] + a1_ref[...])
    logits = jnp.dot(agg, w2_ref[...], preferred_element_type=jnp.float32)
    logits = logits + b2_ref[...]
    m = jnp.max(logits, axis=1, keepdims=True)
    lse = jnp.log(jnp.sum(jnp.exp(logits - m), axis=1, keepdims=True)) + m
    o_ref[...] = logits - lse


_tc3 = pl.pallas_call(
    _tc3_body,
    grid=(_NPAD // _RBLK,),
    in_specs=[
        pl.BlockSpec((_RBLK, _DH), lambda i: (i, 0)),
        pl.BlockSpec((_RBLK, _DH), lambda i: (i, 0)),
        pl.BlockSpec((_RBLK, 1), lambda i: (i, 0)),
        pl.BlockSpec((_RBLK, 1), lambda i: (i, 0)),
        pl.BlockSpec((_DH, _DC), lambda i: (0, 0)),
        pl.BlockSpec((1, _DC), lambda i: (0, 0)),
    ],
    out_specs=pl.BlockSpec((_RBLK, _DC), lambda i: (i, 0)),
    out_shape=jax.ShapeDtypeStruct((_NPAD, _DC), jnp.float32),
)


def kernel(x, edge_index, W1, b1, W2, b2):
    src = edge_index[0].astype(jnp.int32)
    dst = edge_index[1].astype(jnp.int32)
    pad_e = _EPAD - _E
    src_p = jnp.concatenate([src, jnp.zeros((pad_e,), jnp.int32)])
    dst_p = jnp.concatenate([dst, jnp.full((pad_e,), _N, jnp.int32)])
    src_p = src_p.reshape(_NW, _NCH, _CH)
    dst_p = dst_p.reshape(_NW, _NCH, _CH)
    zeros1 = jnp.zeros((_NPAD,), jnp.float32)
    zeros2 = jnp.zeros((_NPAD, _DH), jnp.float32)
    x_p = jnp.pad(x, ((0, _NPAD - _N), (0, 0)))

    degp = _deg(dst_p, zeros1)
    p0 = degp[0][:, None]
    p1 = degp[1][:, None]
    h = _tc1(x_p, W1, p0, p1)
    a = _agg(h, src_p, dst_p, zeros2)
    z = _tc2(a[0], a[1], p0, p1, b1.reshape(1, _DH))
    a2 = _agg(z, src_p, dst_p, zeros2)
    out = _tc3(a2[0], a2[1], p0, p1, W2, b2.reshape(1, _DC))
    return out[:_N]


# speedup vs baseline: 37.9705x; 1.0402x over previous
import functools

import jax
import jax.numpy as jnp
from jax import lax
from jax.experimental import pallas as pl
from jax.experimental.pallas import tpu as pltpu
from jax.experimental.pallas import tpu_sc as plsc

_N = 10000
_E = 320000
_DF = 128
_DH = 16
_DC = 40

_NC = 2
_NS = 16
_NW = _NC * _NS
_CH = 128
_NCH = 80
_EPAD = _NW * _NCH * _CH
_NPAD = 10240
_RPS = _NPAD // _NS
_RBLK = 1024

_MESH = plsc.VectorSubcoreMesh(
    core_axis_name="c", subcore_axis_name="s", num_cores=_NC, num_subcores=_NS
)


def _deg_body(dst_hbm, zero_hbm, out_hbm, dstv, onesv, accum, dsem):
    c = lax.axis_index("c")
    s = lax.axis_index("s")
    wid = s * _NC + c
    rows = pl.ds(s * _RPS, _RPS)
    pltpu.sync_copy(zero_hbm.at[rows], accum.at[rows])
    pltpu.sync_copy(dst_hbm.at[wid], dstv)
    for i in range(_CH // 16):
        onesv[pl.ds(i * 16, 16)] = jnp.ones((16,), jnp.float32)
    plsc.subcore_barrier()

    def step(g, carry):
        descs = [
            pltpu.async_copy(onesv, accum.at[dstv.at[8 * g + b]], dsem.at[b], add=True)
            for b in range(8)
        ]
        for d in descs:
            d.wait()
        return carry

    lax.fori_loop(0, _NCH // 8, step, 0)
    plsc.subcore_barrier()
    pltpu.sync_copy(accum.at[rows], out_hbm.at[c, rows])


_deg = functools.partial(
    pl.kernel,
    out_type=jax.ShapeDtypeStruct((_NC, _NPAD), jnp.float32),
    mesh=_MESH,
    scratch_types=[
        pltpu.VMEM((_NCH, _CH), jnp.int32),
        pltpu.VMEM((_CH,), jnp.float32),
        pltpu.VMEM_SHARED((_NPAD,), jnp.float32),
        pltpu.SemaphoreType.DMA((8,)),
    ],
    compiler_params=pltpu.CompilerParams(use_tc_tiling_on_sc=False),
)(_deg_body)


def _agg_body(h_hbm, src_hbm, dst_hbm, zero_hbm, out_hbm,
              srcv, dstv, msg, accum, gsem, ssem):
    c = lax.axis_index("c")
    s = lax.axis_index("s")
    wid = s * _NC + c
    rows = pl.ds(s * _RPS, _RPS)

    @pl.when(c == 0)
    def _():
        pltpu.sync_copy(h_hbm.at[rows], accum.at[rows])

    @pl.when(c != 0)
    def _():
        pltpu.sync_copy(zero_hbm.at[rows], accum.at[rows])

    pltpu.sync_copy(src_hbm.at[wid], srcv)
    pltpu.sync_copy(dst_hbm.at[wid], dstv)
    plsc.subcore_barrier()

    def step(g, carry):
        base = 8 * g
        gd = [
            pltpu.async_copy(h_hbm.at[srcv.at[base + b]], msg.at[b], gsem.at[b])
            for b in range(8)
        ]
        sd = []
        for b in range(8):
            gd[b].wait()
            sd.append(pltpu.async_copy(
                msg.at[b], accum.at[dstv.at[base + b]], ssem.at[b], add=True))
        for d in sd:
            d.wait()
        return carry

    lax.fori_loop(0, _NCH // 8, step, 0)
    plsc.subcore_barrier()
    pltpu.sync_copy(accum.at[rows], out_hbm.at[c, rows])


_agg = functools.partial(
    pl.kernel,
    out_type=jax.ShapeDtypeStruct((_NC, _NPAD, _DH), jnp.float32),
    mesh=_MESH,
    scratch_types=[
        pltpu.VMEM((_NCH, _CH), jnp.int32),
        pltpu.VMEM((_NCH, _CH), jnp.int32),
        pltpu.VMEM((8, _CH, _DH), jnp.float32),
        pltpu.VMEM_SHARED((_NPAD, _DH), jnp.float32),
        pltpu.SemaphoreType.DMA((8,)),
        pltpu.SemaphoreType.DMA((8,)),
    ],
    compiler_params=pltpu.CompilerParams(use_tc_tiling_on_sc=False),
)(_agg_body)


def _tc1_body(x_ref, w1_ref, p0_ref, p1_ref, h_ref):
    dis = lax.rsqrt(p0_ref[...] + p1_ref[...] + 1.0)
    h = jnp.dot(x_ref[...], w1_ref[...], preferred_element_type=jnp.float32)
    h_ref[...] = h * dis


_tc1 = pl.pallas_call(
    _tc1_body,
    grid=(_NPAD // _RBLK,),
    in_specs=[
        pl.BlockSpec((_RBLK, _DF), lambda i: (i, 0)),
        pl.BlockSpec((_DF, _DH), lambda i: (0, 0)),
        pl.BlockSpec((_RBLK, 1), lambda i: (i, 0)),
        pl.BlockSpec((_RBLK, 1), lambda i: (i, 0)),
    ],
    out_specs=pl.BlockSpec((_RBLK, _DH), lambda i: (i, 0)),
    out_shape=jax.ShapeDtypeStruct((_NPAD, _DH), jnp.float32),
)


def _tc2_body(a0_ref, a1_ref, p0_ref, p1_ref, b1_ref, z_ref):
    dis = lax.rsqrt(p0_ref[...] + p1_ref[...] + 1.0)
    agg = dis * (a0_ref[...] + a1_ref[...]) + b1_ref[...]
    z_ref[...] = dis * jnp.maximum(agg, 0.0)


_tc2 = pl.pallas_call(
    _tc2_body,
    grid=(_NPAD // _RBLK,),
    in_specs=[
        pl.BlockSpec((_RBLK, _DH), lambda i: (i, 0)),
        pl.BlockSpec((_RBLK, _DH), lambda i: (i, 0)),
        pl.BlockSpec((_RBLK, 1), lambda i: (i, 0)),
        pl.BlockSpec((_RBLK, 1), lambda i: (i, 0)),
        pl.BlockSpec((1, _DH), lambda i: (0, 0)),
    ],
    out_specs=pl.BlockSpec((_RBLK, _DH), lambda i: (i, 0)),
    out_shape=jax.ShapeDtypeStruct((_NPAD, _DH), jnp.float32),
)


def _tc3_body(a0_ref, a1_ref, p0_ref, p1_ref, w2_ref, b2_ref, o_ref):
    dis = lax.rsqrt(p0_ref[...] + p1_ref[...] + 1.0)
    agg = dis * (a0_ref[...] + a1_ref[...])
    logits = jnp.dot(agg, w2_ref[...], preferred_element_type=jnp.float32)
    logits = logits + b2_ref[...]
    m = jnp.max(logits, axis=1, keepdims=True)
    lse = jnp.log(jnp.sum(jnp.exp(logits - m), axis=1, keepdims=True)) + m
    o_ref[...] = logits - lse


_tc3 = pl.pallas_call(
    _tc3_body,
    grid=(_NPAD // _RBLK,),
    in_specs=[
        pl.BlockSpec((_RBLK, _DH), lambda i: (i, 0)),
        pl.BlockSpec((_RBLK, _DH), lambda i: (i, 0)),
        pl.BlockSpec((_RBLK, 1), lambda i: (i, 0)),
        pl.BlockSpec((_RBLK, 1), lambda i: (i, 0)),
        pl.BlockSpec((_DH, _DC), lambda i: (0, 0)),
        pl.BlockSpec((1, _DC), lambda i: (0, 0)),
    ],
    out_specs=pl.BlockSpec((_RBLK, _DC), lambda i: (i, 0)),
    out_shape=jax.ShapeDtypeStruct((_NPAD, _DC), jnp.float32),
)


def kernel(x, edge_index, W1, b1, W2, b2):
    src = edge_index[0].astype(jnp.int32)
    dst = edge_index[1].astype(jnp.int32)
    pad_e = _EPAD - _E
    pad_dst = _N + (jnp.arange(pad_e, dtype=jnp.int32) % (_NPAD - _N))
    src_p = jnp.concatenate([src, jnp.zeros((pad_e,), jnp.int32)])
    dst_p = jnp.concatenate([dst, pad_dst])
    src_p = src_p.reshape(_NW, _NCH, _CH)
    dst_p = dst_p.reshape(_NW, _NCH, _CH)
    zeros1 = jnp.zeros((_NPAD,), jnp.float32)
    zeros2 = jnp.zeros((_NPAD, _DH), jnp.float32)
    x_p = jnp.pad(x, ((0, _NPAD - _N), (0, 0)))

    degp = _deg(dst_p, zeros1)
    p0 = degp[0][:, None]
    p1 = degp[1][:, None]
    h = _tc1(x_p, W1, p0, p1)
    a = _agg(h, src_p, dst_p, zeros2)
    z = _tc2(a[0], a[1], p0, p1, b1.reshape(1, _DH))
    a2 = _agg(z, src_p, dst_p, zeros2)
    out = _tc3(a2[0], a2[1], p0, p1, W2, b2.reshape(1, _DC))
    return out[:_N]


# speedup vs baseline: 38.1754x; 1.0054x over previous
import functools

import jax
import jax.numpy as jnp
from jax import lax
from jax.experimental import pallas as pl
from jax.experimental.pallas import tpu as pltpu
from jax.experimental.pallas import tpu_sc as plsc

_N = 10000
_E = 320000
_DF = 128
_DH = 16
_DC = 40

_NC = 2
_NS = 16
_NW = _NC * _NS
_CH = 128
_NCH = 80
_EPAD = _NW * _NCH * _CH
_NPAD = 10240
_RPS = _NPAD // _NS
_RBLK = 1024

_MESH = plsc.VectorSubcoreMesh(
    core_axis_name="c", subcore_axis_name="s", num_cores=_NC, num_subcores=_NS
)


def _deg_body(dst_hbm, zero_hbm, out_hbm, dstv, onesv, accum, dsem):
    c = lax.axis_index("c")
    s = lax.axis_index("s")
    wid = s * _NC + c
    rows = pl.ds(s * _RPS, _RPS)
    pltpu.sync_copy(zero_hbm.at[rows], accum.at[rows])
    pltpu.sync_copy(dst_hbm.at[wid], dstv)
    for i in range(_CH // 16):
        onesv[pl.ds(i * 16, 16)] = jnp.ones((16,), jnp.float32)
    plsc.subcore_barrier()

    def step(g, carry):
        descs = [
            pltpu.async_copy(onesv, accum.at[dstv.at[8 * g + b]], dsem.at[b], add=True)
            for b in range(8)
        ]
        for d in descs:
            d.wait()
        return carry

    lax.fori_loop(0, _NCH // 8, step, 0)
    plsc.subcore_barrier()
    pltpu.sync_copy(accum.at[rows], out_hbm.at[c, rows])


_deg = functools.partial(
    pl.kernel,
    out_type=jax.ShapeDtypeStruct((_NC, _NPAD), jnp.float32),
    mesh=_MESH,
    scratch_types=[
        pltpu.VMEM((_NCH, _CH), jnp.int32),
        pltpu.VMEM((_CH,), jnp.float32),
        pltpu.VMEM_SHARED((_NPAD,), jnp.float32),
        pltpu.SemaphoreType.DMA((8,)),
    ],
    compiler_params=pltpu.CompilerParams(use_tc_tiling_on_sc=False),
)(_deg_body)


def _agg_body(h_hbm, src_hbm, dst_hbm, out_hbm,
              srcv, dstv, msg, accum, gsem, ssem):
    c = lax.axis_index("c")
    s = lax.axis_index("s")
    wid = s * _NC + c
    rows = pl.ds(s * _RPS, _RPS)

    pltpu.sync_copy(h_hbm.at[rows], accum.at[rows])

    pltpu.sync_copy(src_hbm.at[wid], srcv)
    pltpu.sync_copy(dst_hbm.at[wid], dstv)
    plsc.subcore_barrier()

    def step(g, carry):
        base = 8 * g
        gd = [
            pltpu.async_copy(h_hbm.at[srcv.at[base + b]], msg.at[b], gsem.at[b])
            for b in range(8)
        ]
        sd = []
        for b in range(8):
            gd[b].wait()
            sd.append(pltpu.async_copy(
                msg.at[b], accum.at[dstv.at[base + b]], ssem.at[b], add=True))
        for d in sd:
            d.wait()
        return carry

    lax.fori_loop(0, _NCH // 8, step, 0)
    plsc.subcore_barrier()
    pltpu.sync_copy(accum.at[rows], out_hbm.at[c, rows])


_agg = functools.partial(
    pl.kernel,
    out_type=jax.ShapeDtypeStruct((_NC, _NPAD, _DH), jnp.float32),
    mesh=_MESH,
    scratch_types=[
        pltpu.VMEM((_NCH, _CH), jnp.int32),
        pltpu.VMEM((_NCH, _CH), jnp.int32),
        pltpu.VMEM((8, _CH, _DH), jnp.float32),
        pltpu.VMEM_SHARED((_NPAD, _DH), jnp.float32),
        pltpu.SemaphoreType.DMA((8,)),
        pltpu.SemaphoreType.DMA((8,)),
    ],
    compiler_params=pltpu.CompilerParams(use_tc_tiling_on_sc=False),
)(_agg_body)


def _tc1_body(x_ref, w1_ref, p0_ref, p1_ref, h_ref):
    dis = lax.rsqrt(p0_ref[...] + p1_ref[...] + 1.0)
    h = jnp.dot(x_ref[...], w1_ref[...], preferred_element_type=jnp.float32)
    h_ref[...] = h * dis


_tc1 = pl.pallas_call(
    _tc1_body,
    grid=(_NPAD // _RBLK,),
    in_specs=[
        pl.BlockSpec((_RBLK, _DF), lambda i: (i, 0)),
        pl.BlockSpec((_DF, _DH), lambda i: (0, 0)),
        pl.BlockSpec((_RBLK, 1), lambda i: (i, 0)),
        pl.BlockSpec((_RBLK, 1), lambda i: (i, 0)),
    ],
    out_specs=pl.BlockSpec((_RBLK, _DH), lambda i: (i, 0)),
    out_shape=jax.ShapeDtypeStruct((_NPAD, _DH), jnp.float32),
)


def _tc2_body(a0_ref, a1_ref, h_ref, p0_ref, p1_ref, b1_ref, z_ref):
    dis = lax.rsqrt(p0_ref[...] + p1_ref[...] + 1.0)
    agg = dis * (a0_ref[...] + a1_ref[...] - h_ref[...]) + b1_ref[...]
    z_ref[...] = dis * jnp.maximum(agg, 0.0)


_tc2 = pl.pallas_call(
    _tc2_body,
    grid=(_NPAD // _RBLK,),
    in_specs=[
        pl.BlockSpec((_RBLK, _DH), lambda i: (i, 0)),
        pl.BlockSpec((_RBLK, _DH), lambda i: (i, 0)),
        pl.BlockSpec((_RBLK, _DH), lambda i: (i, 0)),
        pl.BlockSpec((_RBLK, 1), lambda i: (i, 0)),
        pl.BlockSpec((_RBLK, 1), lambda i: (i, 0)),
        pl.BlockSpec((1, _DH), lambda i: (0, 0)),
    ],
    out_specs=pl.BlockSpec((_RBLK, _DH), lambda i: (i, 0)),
    out_shape=jax.ShapeDtypeStruct((_NPAD, _DH), jnp.float32),
)


def _tc3_body(a0_ref, a1_ref, z_ref, p0_ref, p1_ref, w2_ref, b2_ref, o_ref):
    dis = lax.rsqrt(p0_ref[...] + p1_ref[...] + 1.0)
    agg = dis * (a0_ref[...] + a1_ref[...] - z_ref[...])
    logits = jnp.dot(agg, w2_ref[...], preferred_element_type=jnp.float32)
    logits = logits + b2_ref[...]
    m = jnp.max(logits, axis=1, keepdims=True)
    lse = jnp.log(jnp.sum(jnp.exp(logits - m), axis=1, keepdims=True)) + m
    o_ref[...] = logits - lse


_tc3 = pl.pallas_call(
    _tc3_body,
    grid=(_NPAD // _RBLK,),
    in_specs=[
        pl.BlockSpec((_RBLK, _DH), lambda i: (i, 0)),
        pl.BlockSpec((_RBLK, _DH), lambda i: (i, 0)),
        pl.BlockSpec((_RBLK, _DH), lambda i: (i, 0)),
        pl.BlockSpec((_RBLK, 1), lambda i: (i, 0)),
        pl.BlockSpec((_RBLK, 1), lambda i: (i, 0)),
        pl.BlockSpec((_DH, _DC), lambda i: (0, 0)),
        pl.BlockSpec((1, _DC), lambda i: (0, 0)),
    ],
    out_specs=pl.BlockSpec((_RBLK, _DC), lambda i: (i, 0)),
    out_shape=jax.ShapeDtypeStruct((_NPAD, _DC), jnp.float32),
)


def kernel(x, edge_index, W1, b1, W2, b2):
    src = edge_index[0].astype(jnp.int32)
    dst = edge_index[1].astype(jnp.int32)
    pad_e = _EPAD - _E
    pad_dst = _N + (jnp.arange(pad_e, dtype=jnp.int32) % (_NPAD - _N))
    src_p = jnp.concatenate([src, jnp.zeros((pad_e,), jnp.int32)])
    dst_p = jnp.concatenate([dst, pad_dst])
    src_p = src_p.reshape(_NW, _NCH, _CH)
    dst_p = dst_p.reshape(_NW, _NCH, _CH)
    zeros1 = jnp.zeros((_NPAD,), jnp.float32)
    x_p = jnp.pad(x, ((0, _NPAD - _N), (0, 0)))

    degp = _deg(dst_p, zeros1)
    p0 = degp[0][:, None]
    p1 = degp[1][:, None]
    h = _tc1(x_p, W1, p0, p1)
    a = _agg(h, src_p, dst_p)
    z = _tc2(a[0], a[1], h, p0, p1, b1.reshape(1, _DH))
    a2 = _agg(z, src_p, dst_p)
    out = _tc3(a2[0], a2[1], z, p0, p1, W2, b2.reshape(1, _DC))
    return out[:_N]


# speedup vs baseline: 53.2516x; 1.3949x over previous
import functools

import jax
import jax.numpy as jnp
from jax import lax
from jax.experimental import pallas as pl
from jax.experimental.pallas import tpu as pltpu
from jax.experimental.pallas import tpu_sc as plsc

_N = 10000
_E = 320000
_DF = 128
_DH = 16
_DC = 40

_NC = 2
_NS = 16
_NW = _NC * _NS
_CH = 128
_NCH = 80
_EPAD = _NW * _NCH * _CH
_NPAD = 10240
_RPS = _NPAD // _NS
_RBLK = 1024

_MESH = plsc.VectorSubcoreMesh(
    core_axis_name="c", subcore_axis_name="s", num_cores=_NC, num_subcores=_NS
)


def _deg_body(dst_hbm, zero_hbm, out_hbm, dstv, onesv, accum, dsem):
    c = lax.axis_index("c")
    s = lax.axis_index("s")
    wid = s * _NC + c
    rows = pl.ds(s * _RPS, _RPS)
    pltpu.sync_copy(zero_hbm.at[rows], accum.at[rows])
    pltpu.sync_copy(dst_hbm.at[wid], dstv)
    for i in range(_CH // 16):
        onesv[pl.ds(i * 16, 16)] = jnp.ones((16,), jnp.float32)
    plsc.subcore_barrier()

    def step(g, carry):
        descs = [
            pltpu.async_copy(onesv, accum.at[dstv.at[8 * g + b]], dsem.at[b], add=True)
            for b in range(8)
        ]
        for d in descs:
            d.wait()
        return carry

    lax.fori_loop(0, _NCH // 8, step, 0)
    plsc.subcore_barrier()
    pltpu.sync_copy(accum.at[rows], out_hbm.at[c, rows])


_deg = functools.partial(
    pl.kernel,
    out_type=jax.ShapeDtypeStruct((_NC, _NPAD), jnp.float32),
    mesh=_MESH,
    scratch_types=[
        pltpu.VMEM((_NCH, _CH), jnp.int32),
        pltpu.VMEM((_CH,), jnp.float32),
        pltpu.VMEM_SHARED((_NPAD,), jnp.float32),
        pltpu.SemaphoreType.DMA((8,)),
    ],
    compiler_params=pltpu.CompilerParams(use_tc_tiling_on_sc=False),
)(_deg_body)


def _agg_body(h_hbm, src_hbm, dst_hbm, out_hbm,
              srcv, dstv, msg, h_sp, accum, gsem, ssem):
    c = lax.axis_index("c")
    s = lax.axis_index("s")
    wid = s * _NC + c
    rows = pl.ds(s * _RPS, _RPS)

    pltpu.sync_copy(h_hbm.at[rows], h_sp.at[rows])
    pltpu.sync_copy(h_hbm.at[rows], accum.at[rows])

    pltpu.sync_copy(src_hbm.at[wid], srcv)
    pltpu.sync_copy(dst_hbm.at[wid], dstv)
    plsc.subcore_barrier()

    def step(g, carry):
        base = 8 * g
        gd = [
            pltpu.async_copy(h_sp.at[srcv.at[base + b]], msg.at[b], gsem.at[b])
            for b in range(8)
        ]
        sd = []
        for b in range(8):
            gd[b].wait()
            sd.append(pltpu.async_copy(
                msg.at[b], accum.at[dstv.at[base + b]], ssem.at[b], add=True))
        for d in sd:
            d.wait()
        return carry

    lax.fori_loop(0, _NCH // 8, step, 0)
    plsc.subcore_barrier()
    pltpu.sync_copy(accum.at[rows], out_hbm.at[c, rows])


_agg = functools.partial(
    pl.kernel,
    out_type=jax.ShapeDtypeStruct((_NC, _NPAD, _DH), jnp.float32),
    mesh=_MESH,
    scratch_types=[
        pltpu.VMEM((_NCH, _CH), jnp.int32),
        pltpu.VMEM((_NCH, _CH), jnp.int32),
        pltpu.VMEM((8, _CH, _DH), jnp.float32),
        pltpu.VMEM_SHARED((_NPAD, _DH), jnp.float32),
        pltpu.VMEM_SHARED((_NPAD, _DH), jnp.float32),
        pltpu.SemaphoreType.DMA((8,)),
        pltpu.SemaphoreType.DMA((8,)),
    ],
    compiler_params=pltpu.CompilerParams(use_tc_tiling_on_sc=False),
)(_agg_body)


def _tc1_body(x_ref, w1_ref, p0_ref, p1_ref, h_ref):
    dis = lax.rsqrt(p0_ref[...] + p1_ref[...] + 1.0)
    h = jnp.dot(x_ref[...], w1_ref[...], preferred_element_type=jnp.float32)
    h_ref[...] = h * dis


_tc1 = pl.pallas_call(
    _tc1_body,
    grid=(_NPAD // _RBLK,),
    in_specs=[
        pl.BlockSpec((_RBLK, _DF), lambda i: (i, 0)),
        pl.BlockSpec((_DF, _DH), lambda i: (0, 0)),
        pl.BlockSpec((_RBLK, 1), lambda i: (i, 0)),
        pl.BlockSpec((_RBLK, 1), lambda i: (i, 0)),
    ],
    out_specs=pl.BlockSpec((_RBLK, _DH), lambda i: (i, 0)),
    out_shape=jax.ShapeDtypeStruct((_NPAD, _DH), jnp.float32),
)


def _tc2_body(a0_ref, a1_ref, h_ref, p0_ref, p1_ref, b1_ref, z_ref):
    dis = lax.rsqrt(p0_ref[...] + p1_ref[...] + 1.0)
    agg = dis * (a0_ref[...] + a1_ref[...] - h_ref[...]) + b1_ref[...]
    z_ref[...] = dis * jnp.maximum(agg, 0.0)


_tc2 = pl.pallas_call(
    _tc2_body,
    grid=(_NPAD // _RBLK,),
    in_specs=[
        pl.BlockSpec((_RBLK, _DH), lambda i: (i, 0)),
        pl.BlockSpec((_RBLK, _DH), lambda i: (i, 0)),
        pl.BlockSpec((_RBLK, _DH), lambda i: (i, 0)),
        pl.BlockSpec((_RBLK, 1), lambda i: (i, 0)),
        pl.BlockSpec((_RBLK, 1), lambda i: (i, 0)),
        pl.BlockSpec((1, _DH), lambda i: (0, 0)),
    ],
    out_specs=pl.BlockSpec((_RBLK, _DH), lambda i: (i, 0)),
    out_shape=jax.ShapeDtypeStruct((_NPAD, _DH), jnp.float32),
)


def _tc3_body(a0_ref, a1_ref, z_ref, p0_ref, p1_ref, w2_ref, b2_ref, o_ref):
    dis = lax.rsqrt(p0_ref[...] + p1_ref[...] + 1.0)
    agg = dis * (a0_ref[...] + a1_ref[...] - z_ref[...])
    logits = jnp.dot(agg, w2_ref[...], preferred_element_type=jnp.float32)
    logits = logits + b2_ref[...]
    m = jnp.max(logits, axis=1, keepdims=True)
    lse = jnp.log(jnp.sum(jnp.exp(logits - m), axis=1, keepdims=True)) + m
    o_ref[...] = logits - lse


_tc3 = pl.pallas_call(
    _tc3_body,
    grid=(_NPAD // _RBLK,),
    in_specs=[
        pl.BlockSpec((_RBLK, _DH), lambda i: (i, 0)),
        pl.BlockSpec((_RBLK, _DH), lambda i: (i, 0)),
        pl.BlockSpec((_RBLK, _DH), lambda i: (i, 0)),
        pl.BlockSpec((_RBLK, 1), lambda i: (i, 0)),
        pl.BlockSpec((_RBLK, 1), lambda i: (i, 0)),
        pl.BlockSpec((_DH, _DC), lambda i: (0, 0)),
        pl.BlockSpec((1, _DC), lambda i: (0, 0)),
    ],
    out_specs=pl.BlockSpec((_RBLK, _DC), lambda i: (i, 0)),
    out_shape=jax.ShapeDtypeStruct((_NPAD, _DC), jnp.float32),
)


def kernel(x, edge_index, W1, b1, W2, b2):
    src = edge_index[0].astype(jnp.int32)
    dst = edge_index[1].astype(jnp.int32)
    pad_e = _EPAD - _E
    pad_dst = _N + (jnp.arange(pad_e, dtype=jnp.int32) % (_NPAD - _N))
    src_p = jnp.concatenate([src, jnp.zeros((pad_e,), jnp.int32)])
    dst_p = jnp.concatenate([dst, pad_dst])
    src_p = src_p.reshape(_NW, _NCH, _CH)
    dst_p = dst_p.reshape(_NW, _NCH, _CH)
    zeros1 = jnp.zeros((_NPAD,), jnp.float32)
    x_p = jnp.pad(x, ((0, _NPAD - _N), (0, 0)))

    degp = _deg(dst_p, zeros1)
    p0 = degp[0][:, None]
    p1 = degp[1][:, None]
    h = _tc1(x_p, W1, p0, p1)
    a = _agg(h, src_p, dst_p)
    z = _tc2(a[0], a[1], h, p0, p1, b1.reshape(1, _DH))
    a2 = _agg(z, src_p, dst_p)
    out = _tc3(a2[0], a2[1], z, p0, p1, W2, b2.reshape(1, _DC))
    return out[:_N]


# speedup vs baseline: 55.6433x; 1.0449x over previous
import functools

import jax
import jax.numpy as jnp
from jax import lax
from jax.experimental import pallas as pl
from jax.experimental.pallas import tpu as pltpu
from jax.experimental.pallas import tpu_sc as plsc

_N = 10000
_E = 320000
_DF = 128
_DH = 16
_DC = 40

_NC = 2
_NS = 16
_NW = _NC * _NS
_CH = 128
_NCH = 80
_EPAD = _NW * _NCH * _CH
_NPAD = 10240
_RPS = _NPAD // _NS
_RBLK = 1024

_MESH = plsc.VectorSubcoreMesh(
    core_axis_name="c", subcore_axis_name="s", num_cores=_NC, num_subcores=_NS
)


def _deg_body(ed_hbm, zero_hbm, out_hbm, dstv, onesv, accum, dsem):
    c = lax.axis_index("c")
    s = lax.axis_index("s")
    wid = s * _NC + c
    rows = pl.ds(s * _RPS, _RPS)
    pltpu.sync_copy(zero_hbm.at[rows], accum.at[rows])
    pltpu.sync_copy(ed_hbm.at[1, wid], dstv)
    for i in range(_CH // 16):
        onesv[pl.ds(i * 16, 16)] = jnp.ones((16,), jnp.float32)
    plsc.subcore_barrier()

    def step(g, carry):
        descs = [
            pltpu.async_copy(onesv, accum.at[dstv.at[8 * g + b]], dsem.at[b], add=True)
            for b in range(8)
        ]
        for d in descs:
            d.wait()
        return carry

    lax.fori_loop(0, _NCH // 8, step, 0)
    plsc.subcore_barrier()
    pltpu.sync_copy(accum.at[rows], out_hbm.at[c, rows])


_deg = functools.partial(
    pl.kernel,
    out_type=jax.ShapeDtypeStruct((_NC, _NPAD), jnp.float32),
    mesh=_MESH,
    scratch_types=[
        pltpu.VMEM((_NCH, _CH), jnp.int32),
        pltpu.VMEM((_CH,), jnp.float32),
        pltpu.VMEM_SHARED((_NPAD,), jnp.float32),
        pltpu.SemaphoreType.DMA((8,)),
    ],
    compiler_params=pltpu.CompilerParams(use_tc_tiling_on_sc=False),
)(_deg_body)


def _agg_body(h_hbm, ed_hbm, out_hbm,
              srcv, dstv, msg, h_sp, accum, gsem, ssem):
    c = lax.axis_index("c")
    s = lax.axis_index("s")
    wid = s * _NC + c
    rows = pl.ds(s * _RPS, _RPS)

    pltpu.sync_copy(h_hbm.at[rows], h_sp.at[rows])
    pltpu.sync_copy(h_hbm.at[rows], accum.at[rows])

    pltpu.sync_copy(ed_hbm.at[0, wid], srcv)
    pltpu.sync_copy(ed_hbm.at[1, wid], dstv)
    plsc.subcore_barrier()

    def step(g, carry):
        base = 8 * g
        gd = [
            pltpu.async_copy(h_sp.at[srcv.at[base + b]], msg.at[b], gsem.at[b])
            for b in range(8)
        ]
        sd = []
        for b in range(8):
            gd[b].wait()
            sd.append(pltpu.async_copy(
                msg.at[b], accum.at[dstv.at[base + b]], ssem.at[b], add=True))
        for d in sd:
            d.wait()
        return carry

    lax.fori_loop(0, _NCH // 8, step, 0)
    plsc.subcore_barrier()
    pltpu.sync_copy(accum.at[rows], out_hbm.at[c, rows])


_agg = functools.partial(
    pl.kernel,
    out_type=jax.ShapeDtypeStruct((_NC, _NPAD, _DH), jnp.float32),
    mesh=_MESH,
    scratch_types=[
        pltpu.VMEM((_NCH, _CH), jnp.int32),
        pltpu.VMEM((_NCH, _CH), jnp.int32),
        pltpu.VMEM((8, _CH, _DH), jnp.float32),
        pltpu.VMEM_SHARED((_NPAD, _DH), jnp.float32),
        pltpu.VMEM_SHARED((_NPAD, _DH), jnp.float32),
        pltpu.SemaphoreType.DMA((8,)),
        pltpu.SemaphoreType.DMA((8,)),
    ],
    compiler_params=pltpu.CompilerParams(use_tc_tiling_on_sc=False),
)(_agg_body)


def _tc1_body(x_ref, w1_ref, p0_ref, p1_ref, h_ref):
    dis = lax.rsqrt(p0_ref[...] + p1_ref[...] + 1.0)
    h = jnp.dot(x_ref[...], w1_ref[...], preferred_element_type=jnp.float32)
    h_ref[...] = h * dis


_RB1 = 1000

_tc1 = pl.pallas_call(
    _tc1_body,
    grid=(_N // _RB1,),
    in_specs=[
        pl.BlockSpec((_RB1, _DF), lambda i: (i, 0)),
        pl.BlockSpec((_DF, _DH), lambda i: (0, 0)),
        pl.BlockSpec((_RB1, 1), lambda i: (i, 0)),
        pl.BlockSpec((_RB1, 1), lambda i: (i, 0)),
    ],
    out_specs=pl.BlockSpec((_RB1, _DH), lambda i: (i, 0)),
    out_shape=jax.ShapeDtypeStruct((_NPAD, _DH), jnp.float32),
)


def _tc2_body(a0_ref, a1_ref, h_ref, p0_ref, p1_ref, b1_ref, z_ref):
    dis = lax.rsqrt(p0_ref[...] + p1_ref[...] + 1.0)
    agg = dis * (a0_ref[...] + a1_ref[...] - h_ref[...]) + b1_ref[...]
    z_ref[...] = dis * jnp.maximum(agg, 0.0)


_tc2 = pl.pallas_call(
    _tc2_body,
    grid=(_NPAD // _RBLK,),
    in_specs=[
        pl.BlockSpec((_RBLK, _DH), lambda i: (i, 0)),
        pl.BlockSpec((_RBLK, _DH), lambda i: (i, 0)),
        pl.BlockSpec((_RBLK, _DH), lambda i: (i, 0)),
        pl.BlockSpec((_RBLK, 1), lambda i: (i, 0)),
        pl.BlockSpec((_RBLK, 1), lambda i: (i, 0)),
        pl.BlockSpec((1, _DH), lambda i: (0, 0)),
    ],
    out_specs=pl.BlockSpec((_RBLK, _DH), lambda i: (i, 0)),
    out_shape=jax.ShapeDtypeStruct((_NPAD, _DH), jnp.float32),
)


def _tc3_body(a0_ref, a1_ref, z_ref, p0_ref, p1_ref, w2_ref, b2_ref, o_ref):
    dis = lax.rsqrt(p0_ref[...] + p1_ref[...] + 1.0)
    agg = dis * (a0_ref[...] + a1_ref[...] - z_ref[...])
    logits = jnp.dot(agg, w2_ref[...], preferred_element_type=jnp.float32)
    logits = logits + b2_ref[...]
    m = jnp.max(logits, axis=1, keepdims=True)
    lse = jnp.log(jnp.sum(jnp.exp(logits - m), axis=1, keepdims=True)) + m
    o_ref[...] = logits - lse


_tc3 = pl.pallas_call(
    _tc3_body,
    grid=(_NPAD // _RBLK,),
    in_specs=[
        pl.BlockSpec((_RBLK, _DH), lambda i: (i, 0)),
        pl.BlockSpec((_RBLK, _DH), lambda i: (i, 0)),
        pl.BlockSpec((_RBLK, _DH), lambda i: (i, 0)),
        pl.BlockSpec((_RBLK, 1), lambda i: (i, 0)),
        pl.BlockSpec((_RBLK, 1), lambda i: (i, 0)),
        pl.BlockSpec((_DH, _DC), lambda i: (0, 0)),
        pl.BlockSpec((1, _DC), lambda i: (0, 0)),
    ],
    out_specs=pl.BlockSpec((_RBLK, _DC), lambda i: (i, 0)),
    out_shape=jax.ShapeDtypeStruct((_NPAD, _DC), jnp.float32),
)


def kernel(x, edge_index, W1, b1, W2, b2):
    pad_e = _EPAD - _E
    pad_dst = _N + (jnp.arange(pad_e, dtype=jnp.int32) % (_NPAD - _N))
    pad_blk = jnp.stack([jnp.zeros((pad_e,), jnp.int32), pad_dst])
    ed = jnp.concatenate([edge_index.astype(jnp.int32), pad_blk], axis=1)
    ed = ed.reshape(2, _NW, _NCH, _CH)
    zeros1 = jnp.zeros((_NPAD,), jnp.float32)

    degp = _deg(ed, zeros1)
    p0 = degp[0][:, None]
    p1 = degp[1][:, None]
    h = _tc1(x, W1, p0, p1)
    a = _agg(h, ed)
    z = _tc2(a[0], a[1], h, p0, p1, b1.reshape(1, _DH))
    a2 = _agg(z, ed)
    out = _tc3(a2[0], a2[1], z, p0, p1, W2, b2.reshape(1, _DC))
    return out[:_N]


# speedup vs baseline: 71.0278x; 1.2765x over previous
import functools

import jax
import jax.numpy as jnp
import numpy as np
from jax import lax
from jax.experimental import pallas as pl
from jax.experimental.pallas import tpu as pltpu
from jax.experimental.pallas import tpu_sc as plsc

_N = 10000
_E = 320000
_DF = 128
_DH = 16
_DC = 40

_NC = 2
_NS = 16
_NW = _NC * _NS
_CH = 128
_NCH = 80
_EPAD = _NW * _NCH * _CH
_NPAD = 10240
_RPS = _NPAD // _NS
_RBLK = 1024

_MESH = plsc.VectorSubcoreMesh(
    core_axis_name="c", subcore_axis_name="s", num_cores=_NC, num_subcores=_NS
)


def _deg_body(ed_hbm, zero_hbm, out_hbm, oute_hbm, dstv, onesv, degv, expv,
              accum, dsem):
    c = lax.axis_index("c")
    s = lax.axis_index("s")
    wid = s * _NC + c
    rows = pl.ds(s * _RPS, _RPS)
    pltpu.sync_copy(zero_hbm.at[rows], accum.at[rows])
    pltpu.sync_copy(ed_hbm.at[1, wid], dstv)
    for i in range(_CH // 16):
        onesv[pl.ds(i * 16, 16)] = jnp.ones((16,), jnp.float32)
    plsc.subcore_barrier()

    def step(g, carry):
        descs = [
            pltpu.async_copy(onesv, accum.at[dstv.at[8 * g + b]], dsem.at[b], add=True)
            for b in range(8)
        ]
        for d in descs:
            d.wait()
        return carry

    lax.fori_loop(0, _NCH // 8, step, 0)
    plsc.subcore_barrier()
    pltpu.sync_copy(accum.at[rows], out_hbm.at[c, rows])
    pltpu.sync_copy(accum.at[rows], degv)

    def expand(k, carry):
        v = degv[pl.ds(16 * k, 16)]
        for j in range(16):
            expv[16 * k + j] = jnp.full((_DH,), v[j], jnp.float32)
        return carry

    lax.fori_loop(0, _RPS // 16, expand, 0)
    pltpu.sync_copy(expv, oute_hbm.at[c, rows])


_deg = functools.partial(
    pl.kernel,
    out_type=(
        jax.ShapeDtypeStruct((_NC, _NPAD), jnp.float32),
        jax.ShapeDtypeStruct((_NC, _NPAD, _DH), jnp.float32),
    ),
    mesh=_MESH,
    scratch_types=[
        pltpu.VMEM((_NCH, _CH), jnp.int32),
        pltpu.VMEM((_CH,), jnp.float32),
        pltpu.VMEM((_RPS,), jnp.float32),
        pltpu.VMEM((_RPS, _DH), jnp.float32),
        pltpu.VMEM_SHARED((_NPAD,), jnp.float32),
        pltpu.SemaphoreType.DMA((8,)),
    ],
    compiler_params=pltpu.CompilerParams(use_tc_tiling_on_sc=False),
)(_deg_body)


def _agg_body(h_hbm, ed_hbm, out_hbm,
              srcv, dstv, msg, h_sp, accum, gsem, ssem):
    c = lax.axis_index("c")
    s = lax.axis_index("s")
    wid = s * _NC + c
    rows = pl.ds(s * _RPS, _RPS)

    pltpu.sync_copy(h_hbm.at[rows], h_sp.at[rows])
    pltpu.sync_copy(h_hbm.at[rows], accum.at[rows])

    pltpu.sync_copy(ed_hbm.at[0, wid], srcv)
    pltpu.sync_copy(ed_hbm.at[1, wid], dstv)
    plsc.subcore_barrier()

    def step(g, carry):
        base = 8 * g
        gd = [
            pltpu.async_copy(h_sp.at[srcv.at[base + b]], msg.at[b], gsem.at[b])
            for b in range(8)
        ]
        sd = []
        for b in range(8):
            gd[b].wait()
            sd.append(pltpu.async_copy(
                msg.at[b], accum.at[dstv.at[base + b]], ssem.at[b], add=True))
        for d in sd:
            d.wait()
        return carry

    lax.fori_loop(0, _NCH // 8, step, 0)
    plsc.subcore_barrier()
    pltpu.sync_copy(accum.at[rows], out_hbm.at[c, rows])


_agg = functools.partial(
    pl.kernel,
    out_type=jax.ShapeDtypeStruct((_NC, _NPAD, _DH), jnp.float32),
    mesh=_MESH,
    scratch_types=[
        pltpu.VMEM((_NCH, _CH), jnp.int32),
        pltpu.VMEM((_NCH, _CH), jnp.int32),
        pltpu.VMEM((8, _CH, _DH), jnp.float32),
        pltpu.VMEM_SHARED((_NPAD, _DH), jnp.float32),
        pltpu.VMEM_SHARED((_NPAD, _DH), jnp.float32),
        pltpu.SemaphoreType.DMA((8,)),
        pltpu.SemaphoreType.DMA((8,)),
    ],
    compiler_params=pltpu.CompilerParams(use_tc_tiling_on_sc=False),
)(_agg_body)


def _tc1_body(x_ref, w1_ref, p0_ref, p1_ref, h_ref):
    dis = lax.rsqrt(p0_ref[...] + p1_ref[...] + 1.0)
    h = jnp.dot(x_ref[...], w1_ref[...], preferred_element_type=jnp.float32)
    h_ref[...] = h * dis


_RB1 = 1000

_tc1 = pl.pallas_call(
    _tc1_body,
    grid=(_N // _RB1,),
    in_specs=[
        pl.BlockSpec((_RB1, _DF), lambda i: (i, 0)),
        pl.BlockSpec((_DF, _DH), lambda i: (0, 0)),
        pl.BlockSpec((_RB1, 1), lambda i: (i, 0)),
        pl.BlockSpec((_RB1, 1), lambda i: (i, 0)),
    ],
    out_specs=pl.BlockSpec((_RB1, _DH), lambda i: (i, 0)),
    out_shape=jax.ShapeDtypeStruct((_NPAD, _DH), jnp.float32),
)


_NF = _NPAD // 8
_FBLK = _NF // 10
_DC8 = 8 * _DC


def _tc2f_body(a0_ref, a1_ref, h_ref, e0_ref, e1_ref, b1_ref, z_ref):
    dis = lax.rsqrt(e0_ref[...] + e1_ref[...] + 1.0)
    agg = dis * (a0_ref[...] + a1_ref[...] - h_ref[...]) + b1_ref[...]
    z_ref[...] = dis * jnp.maximum(agg, 0.0)


_tc2f = pl.pallas_call(
    _tc2f_body,
    grid=(_NF // _FBLK,),
    in_specs=[
        pl.BlockSpec((None, _FBLK, 128), lambda i: (0, i, 0)),
        pl.BlockSpec((None, _FBLK, 128), lambda i: (1, i, 0)),
        pl.BlockSpec((_FBLK, 128), lambda i: (i, 0)),
        pl.BlockSpec((None, _FBLK, 128), lambda i: (0, i, 0)),
        pl.BlockSpec((None, _FBLK, 128), lambda i: (1, i, 0)),
        pl.BlockSpec((1, 128), lambda i: (0, 0)),
    ],
    out_specs=pl.BlockSpec((_FBLK, 128), lambda i: (i, 0)),
    out_shape=jax.ShapeDtypeStruct((_NF, 128), jnp.float32),
)


def _tc3f_body(a0_ref, a1_ref, z_ref, e0_ref, e1_ref, bd_ref, seg_ref,
               b2_ref, o_ref):
    dis = lax.rsqrt(e0_ref[...] + e1_ref[...] + 1.0)
    t = dis * (a0_ref[...] + a1_ref[...] - z_ref[...])
    logits = jnp.dot(t, bd_ref[...], preferred_element_type=jnp.float32)
    logits = logits + b2_ref[...]
    m = jnp.max(logits, axis=1, keepdims=True)
    e = jnp.exp(logits - m)
    s = jnp.dot(e, seg_ref[...], preferred_element_type=jnp.float32)
    o_ref[...] = logits - m - jnp.log(s)


_tc3f = pl.pallas_call(
    _tc3f_body,
    grid=(_NF // _FBLK,),
    in_specs=[
        pl.BlockSpec((None, _FBLK, 128), lambda i: (0, i, 0)),
        pl.BlockSpec((None, _FBLK, 128), lambda i: (1, i, 0)),
        pl.BlockSpec((_FBLK, 128), lambda i: (i, 0)),
        pl.BlockSpec((None, _FBLK, 128), lambda i: (0, i, 0)),
        pl.BlockSpec((None, _FBLK, 128), lambda i: (1, i, 0)),
        pl.BlockSpec((128, _DC8), lambda i: (0, 0)),
        pl.BlockSpec((_DC8, _DC8), lambda i: (0, 0)),
        pl.BlockSpec((1, _DC8), lambda i: (0, 0)),
    ],
    out_specs=pl.BlockSpec((_FBLK, _DC8), lambda i: (i, 0)),
    out_shape=jax.ShapeDtypeStruct((_NF, _DC8), jnp.float32),
)

_SEG = np.kron(np.eye(8, dtype=np.float32), np.ones((_DC, _DC), np.float32))


def kernel(x, edge_index, W1, b1, W2, b2):
    pad_e = _EPAD - _E
    pad_dst = _N + (jnp.arange(pad_e, dtype=jnp.int32) % (_NPAD - _N))
    pad_blk = jnp.stack([jnp.zeros((pad_e,), jnp.int32), pad_dst])
    ed = jnp.concatenate([edge_index.astype(jnp.int32), pad_blk], axis=1)
    ed = ed.reshape(2, _NW, _NCH, _CH)
    zeros1 = jnp.zeros((_NPAD,), jnp.float32)

    degp, dege = _deg(ed, zeros1)
    p0 = degp[0][:, None]
    p1 = degp[1][:, None]
    degef = dege.reshape(_NC, _NF, 128)
    h = _tc1(x, W1, p0, p1)
    hf = h.reshape(_NF, 128)
    a = _agg(h, ed)
    af = a.reshape(_NC, _NF, 128)
    zf = _tc2f(af, af, hf, degef, degef, jnp.tile(b1, 8).reshape(1, 128))
    a2 = _agg(zf.reshape(_NPAD, _DH), ed)
    a2f = a2.reshape(_NC, _NF, 128)
    bd = jnp.kron(jnp.eye(8, dtype=W2.dtype), W2)
    out = _tc3f(a2f, a2f, zf, degef, degef, bd,
                jnp.asarray(_SEG), jnp.tile(b2, 8).reshape(1, _DC8))
    return out.reshape(_NPAD, _DC)[:_N]


# speedup vs baseline: 78.9719x; 1.1118x over previous
import functools

import jax
import jax.numpy as jnp
import numpy as np
from jax import lax
from jax.experimental import pallas as pl
from jax.experimental.pallas import tpu as pltpu
from jax.experimental.pallas import tpu_sc as plsc

_N = 10000
_E = 320000
_DF = 128
_DH = 16
_DC = 40

_NC = 2
_NS = 16
_NW = _NC * _NS
_CH = 128
_NCH = 80
_EPAD = _NW * _NCH * _CH
_NPAD = 10240
_RPS = _NPAD // _NS
_RBLK = 1024

_MESH = plsc.VectorSubcoreMesh(
    core_axis_name="c", subcore_axis_name="s", num_cores=_NC, num_subcores=_NS
)


def _deg_body(ed_hbm, zero_hbm, oute_hbm, dstv, onesv, degv, expv,
              accum, dsem):
    c = lax.axis_index("c")
    s = lax.axis_index("s")
    wid = s * _NC + c
    rows = pl.ds(s * _RPS, _RPS)
    pltpu.sync_copy(zero_hbm.at[rows], accum.at[rows])
    pltpu.sync_copy(ed_hbm.at[1, wid], dstv)
    for i in range(_CH // 16):
        onesv[pl.ds(i * 16, 16)] = jnp.ones((16,), jnp.float32)
    plsc.subcore_barrier()

    def step(g, carry):
        descs = [
            pltpu.async_copy(onesv, accum.at[dstv.at[8 * g + b]], dsem.at[b], add=True)
            for b in range(8)
        ]
        for d in descs:
            d.wait()
        return carry

    lax.fori_loop(0, _NCH // 8, step, 0)
    plsc.subcore_barrier()
    pltpu.sync_copy(accum.at[rows], degv)

    def expand(k, carry):
        v = degv[pl.ds(16 * k, 16)]
        for j in range(16):
            expv[16 * k + j] = jnp.full((_DH,), v[j], jnp.float32)
        return carry

    lax.fori_loop(0, _RPS // 16, expand, 0)
    pltpu.sync_copy(expv, oute_hbm.at[c, rows])


_deg = functools.partial(
    pl.kernel,
    out_type=jax.ShapeDtypeStruct((_NC, _NPAD, _DH), jnp.float32),
    mesh=_MESH,
    scratch_types=[
        pltpu.VMEM((_NCH, _CH), jnp.int32),
        pltpu.VMEM((_CH,), jnp.float32),
        pltpu.VMEM((_RPS,), jnp.float32),
        pltpu.VMEM((_RPS, _DH), jnp.float32),
        pltpu.VMEM_SHARED((_NPAD,), jnp.float32),
        pltpu.SemaphoreType.DMA((8,)),
    ],
    compiler_params=pltpu.CompilerParams(use_tc_tiling_on_sc=False),
)(_deg_body)


def _agg_body(h_hbm, ed_hbm, out_hbm,
              srcv, dstv, msg, h_sp, accum, gsem, ssem):
    c = lax.axis_index("c")
    s = lax.axis_index("s")
    wid = s * _NC + c
    rows = pl.ds(s * _RPS, _RPS)

    pltpu.sync_copy(h_hbm.at[rows], h_sp.at[rows])
    pltpu.sync_copy(h_hbm.at[rows], accum.at[rows])

    pltpu.sync_copy(ed_hbm.at[0, wid], srcv)
    pltpu.sync_copy(ed_hbm.at[1, wid], dstv)
    plsc.subcore_barrier()

    def step(g, carry):
        base = 8 * g
        gd = [
            pltpu.async_copy(h_sp.at[srcv.at[base + b]], msg.at[b], gsem.at[b])
            for b in range(8)
        ]
        sd = []
        for b in range(8):
            gd[b].wait()
            sd.append(pltpu.async_copy(
                msg.at[b], accum.at[dstv.at[base + b]], ssem.at[b], add=True))
        for d in sd:
            d.wait()
        return carry

    lax.fori_loop(0, _NCH // 8, step, 0)
    plsc.subcore_barrier()
    pltpu.sync_copy(accum.at[rows], out_hbm.at[c, rows])


_agg = functools.partial(
    pl.kernel,
    out_type=jax.ShapeDtypeStruct((_NC, _NPAD, _DH), jnp.float32),
    mesh=_MESH,
    scratch_types=[
        pltpu.VMEM((_NCH, _CH), jnp.int32),
        pltpu.VMEM((_NCH, _CH), jnp.int32),
        pltpu.VMEM((8, _CH, _DH), jnp.float32),
        pltpu.VMEM_SHARED((_NPAD, _DH), jnp.float32),
        pltpu.VMEM_SHARED((_NPAD, _DH), jnp.float32),
        pltpu.SemaphoreType.DMA((8,)),
        pltpu.SemaphoreType.DMA((8,)),
    ],
    compiler_params=pltpu.CompilerParams(use_tc_tiling_on_sc=False),
)(_agg_body)


def _tc1a_body(x_ref, w1k_ref, o_ref):
    o_ref[...] = jnp.dot(x_ref[...], w1k_ref[...],
                         preferred_element_type=jnp.float32)


_tc1a = pl.pallas_call(
    _tc1a_body,
    grid=(1,),
    in_specs=[
        pl.BlockSpec((10240 // 8, 8 * _DF), lambda i: (0, 0)),
        pl.BlockSpec((8 * _DF, 128), lambda i: (0, 0)),
    ],
    out_specs=pl.BlockSpec((10240 // 8, 128), lambda i: (0, 0)),
    out_shape=jax.ShapeDtypeStruct((10240 // 8, 128), jnp.float32),
)


def _tc1b_body(xw_ref, e0_ref, e1_ref, h_ref):
    dis = lax.rsqrt(e0_ref[...] + e1_ref[...] + 1.0)
    h_ref[...] = xw_ref[...] * dis


_NF = _NPAD // 8
_FBLK = _NF // 10
_DC8 = 8 * _DC


_tc1b = pl.pallas_call(
    _tc1b_body,
    grid=(10,),
    in_specs=[
        pl.BlockSpec((128, 128), lambda i: (i, 0)),
        pl.BlockSpec((None, 128, 128), lambda i: (0, i, 0)),
        pl.BlockSpec((None, 128, 128), lambda i: (1, i, 0)),
    ],
    out_specs=pl.BlockSpec((128, 128), lambda i: (i, 0)),
    out_shape=jax.ShapeDtypeStruct((10240 // 8, 128), jnp.float32),
)


def _tc2f_body(a0_ref, a1_ref, h_ref, e0_ref, e1_ref, b1_ref, z_ref):
    dis = lax.rsqrt(e0_ref[...] + e1_ref[...] + 1.0)
    agg = dis * (a0_ref[...] + a1_ref[...] - h_ref[...]) + b1_ref[...]
    z_ref[...] = dis * jnp.maximum(agg, 0.0)


_tc2f = pl.pallas_call(
    _tc2f_body,
    grid=(_NF // _FBLK,),
    in_specs=[
        pl.BlockSpec((None, _FBLK, 128), lambda i: (0, i, 0)),
        pl.BlockSpec((None, _FBLK, 128), lambda i: (1, i, 0)),
        pl.BlockSpec((_FBLK, 128), lambda i: (i, 0)),
        pl.BlockSpec((None, _FBLK, 128), lambda i: (0, i, 0)),
        pl.BlockSpec((None, _FBLK, 128), lambda i: (1, i, 0)),
        pl.BlockSpec((1, 128), lambda i: (0, 0)),
    ],
    out_specs=pl.BlockSpec((_FBLK, 128), lambda i: (i, 0)),
    out_shape=jax.ShapeDtypeStruct((_NF, 128), jnp.float32),
)


def _tc3f_body(a0_ref, a1_ref, z_ref, e0_ref, e1_ref, bd_ref, seg_ref,
               b2_ref, o_ref):
    dis = lax.rsqrt(e0_ref[...] + e1_ref[...] + 1.0)
    t = dis * (a0_ref[...] + a1_ref[...] - z_ref[...])
    logits = jnp.dot(t, bd_ref[...], preferred_element_type=jnp.float32)
    logits = logits + b2_ref[...]
    m = jnp.max(logits, axis=1, keepdims=True)
    e = jnp.exp(logits - m)
    s = jnp.dot(e, seg_ref[...], preferred_element_type=jnp.float32)
    o_ref[...] = logits - m - jnp.log(s)


_tc3f = pl.pallas_call(
    _tc3f_body,
    grid=(_NF // _FBLK,),
    in_specs=[
        pl.BlockSpec((None, _FBLK, 128), lambda i: (0, i, 0)),
        pl.BlockSpec((None, _FBLK, 128), lambda i: (1, i, 0)),
        pl.BlockSpec((_FBLK, 128), lambda i: (i, 0)),
        pl.BlockSpec((None, _FBLK, 128), lambda i: (0, i, 0)),
        pl.BlockSpec((None, _FBLK, 128), lambda i: (1, i, 0)),
        pl.BlockSpec((128, _DC8), lambda i: (0, 0)),
        pl.BlockSpec((_DC8, _DC8), lambda i: (0, 0)),
        pl.BlockSpec((1, _DC8), lambda i: (0, 0)),
    ],
    out_specs=pl.BlockSpec((_FBLK, _DC8), lambda i: (i, 0)),
    out_shape=jax.ShapeDtypeStruct((_NF, _DC8), jnp.float32),
)

_SEG = np.kron(np.eye(8, dtype=np.float32), np.ones((_DC, _DC), np.float32))


def kernel(x, edge_index, W1, b1, W2, b2):
    pad_e = _EPAD - _E
    pad_dst = _N + (jnp.arange(pad_e, dtype=jnp.int32) % (_NPAD - _N))
    pad_blk = jnp.stack([jnp.zeros((pad_e,), jnp.int32), pad_dst])
    ed = jnp.concatenate([edge_index.astype(jnp.int32), pad_blk], axis=1)
    ed = ed.reshape(2, _NW, _NCH, _CH)
    zeros1 = jnp.zeros((_NPAD,), jnp.float32)

    dege = _deg(ed, zeros1)
    degef = dege.reshape(_NC, _NF, 128)
    x_p = jnp.pad(x, ((0, _NPAD - _N), (0, 0))).reshape(_NF, 8 * _DF)
    w1k = jnp.kron(jnp.eye(8, dtype=W1.dtype), W1)
    xw = _tc1a(x_p, w1k)
    hf = _tc1b(xw, degef, degef)
    h = hf.reshape(_NPAD, _DH)
    a = _agg(h, ed)
    af = a.reshape(_NC, _NF, 128)
    zf = _tc2f(af, af, hf, degef, degef, jnp.tile(b1, 8).reshape(1, 128))
    a2 = _agg(zf.reshape(_NPAD, _DH), ed)
    a2f = a2.reshape(_NC, _NF, 128)
    bd = jnp.kron(jnp.eye(8, dtype=W2.dtype), W2)
    out = _tc3f(a2f, a2f, zf, degef, degef, bd,
                jnp.asarray(_SEG), jnp.tile(b2, 8).reshape(1, _DC8))
    return out.reshape(_NPAD, _DC)[:_N]


# speedup vs baseline: 82.1038x; 1.0397x over previous
import functools

import jax
import jax.numpy as jnp
import numpy as np
from jax import lax
from jax.experimental import pallas as pl
from jax.experimental.pallas import tpu as pltpu
from jax.experimental.pallas import tpu_sc as plsc

_N = 10000
_E = 320000
_DF = 128
_DH = 16
_DC = 40

_NC = 2
_NS = 16
_NW = _NC * _NS
_CH = 128
_NCH = 80
_EPAD = _NW * _NCH * _CH
_NPAD = 10240
_RPS = _NPAD // _NS
_RBLK = 1024

_MESH = plsc.VectorSubcoreMesh(
    core_axis_name="c", subcore_axis_name="s", num_cores=_NC, num_subcores=_NS
)


def _deg_body(ed_hbm, zero_hbm, oute_hbm, dstv, onesv, degv, expv,
              accum, dsem):
    c = lax.axis_index("c")
    s = lax.axis_index("s")
    wid = s * _NC + c
    rows = pl.ds(s * _RPS, _RPS)
    pltpu.sync_copy(zero_hbm.at[rows], accum.at[rows])
    pltpu.sync_copy(ed_hbm.at[1, wid], dstv)
    for i in range(_CH // 16):
        onesv[pl.ds(i * 16, 16)] = jnp.ones((16,), jnp.float32)
    plsc.subcore_barrier()

    def step(g, carry):
        descs = [
            pltpu.async_copy(onesv, accum.at[dstv.at[8 * g + b]], dsem.at[b],
                             add=True)
            for b in range(8)
        ]
        for d in descs:
            d.wait()
        return carry

    lax.fori_loop(0, _NCH // 8, step, 0)
    plsc.subcore_barrier()
    pltpu.sync_copy(accum.at[rows], degv)

    def expand(k, carry):
        v = degv[pl.ds(16 * k, 16)]
        for j in range(16):
            expv[16 * k + j] = jnp.full((_DH,), v[j], jnp.float32)
        return carry

    lax.fori_loop(0, _RPS // 16, expand, 0)
    pltpu.sync_copy(expv, oute_hbm.at[c, rows])


_deg = functools.partial(
    pl.kernel,
    out_type=jax.ShapeDtypeStruct((_NC, _NPAD, _DH), jnp.float32),
    mesh=_MESH,
    scratch_types=[
        pltpu.VMEM((_NCH, _CH), jnp.int32),
        pltpu.VMEM((_CH,), jnp.float32),
        pltpu.VMEM((_RPS,), jnp.float32),
        pltpu.VMEM((_RPS, _DH), jnp.float32),
        pltpu.VMEM_SHARED((_NPAD,), jnp.float32),
        pltpu.SemaphoreType.DMA((8,)),
    ],
    compiler_params=pltpu.CompilerParams(use_tc_tiling_on_sc=False),
)(_deg_body)


def _agg_body(h_hbm, ed_hbm, out_hbm,
              srcv, dstv, msg, h_sp, accum, gsem, ssem):
    c = lax.axis_index("c")
    s = lax.axis_index("s")
    wid = s * _NC + c
    rows = pl.ds(s * _RPS, _RPS)

    pltpu.sync_copy(h_hbm.at[rows], h_sp.at[rows])
    pltpu.sync_copy(h_hbm.at[rows], accum.at[rows])

    pltpu.sync_copy(ed_hbm.at[0, wid], srcv)
    pltpu.sync_copy(ed_hbm.at[1, wid], dstv)
    plsc.subcore_barrier()

    def step(g, carry):
        base = 8 * g
        gd = [
            pltpu.async_copy(h_sp.at[srcv.at[base + b]], msg.at[b], gsem.at[b])
            for b in range(8)
        ]
        sd = []
        for b in range(8):
            gd[b].wait()
            sd.append(pltpu.async_copy(
                msg.at[b], accum.at[dstv.at[base + b]], ssem.at[b], add=True))
        for d in sd:
            d.wait()
        return carry

    lax.fori_loop(0, _NCH // 8, step, 0)
    plsc.subcore_barrier()
    pltpu.sync_copy(accum.at[rows], out_hbm.at[c, rows])


_agg = functools.partial(
    pl.kernel,
    out_type=jax.ShapeDtypeStruct((_NC, _NPAD, _DH), jnp.float32),
    mesh=_MESH,
    scratch_types=[
        pltpu.VMEM((_NCH, _CH), jnp.int32),
        pltpu.VMEM((_NCH, _CH), jnp.int32),
        pltpu.VMEM((8, _CH, _DH), jnp.float32),
        pltpu.VMEM_SHARED((_NPAD, _DH), jnp.float32),
        pltpu.VMEM_SHARED((_NPAD, _DH), jnp.float32),
        pltpu.SemaphoreType.DMA((8,)),
        pltpu.SemaphoreType.DMA((8,)),
    ],
    compiler_params=pltpu.CompilerParams(use_tc_tiling_on_sc=False),
)(_agg_body)


def _tc1a_body(x_ref, w1k_ref, o_ref):
    o_ref[...] = jnp.dot(x_ref[...], w1k_ref[...],
                         preferred_element_type=jnp.float32)


_tc1a = pl.pallas_call(
    _tc1a_body,
    grid=(1,),
    in_specs=[
        pl.BlockSpec((10240 // 8, 8 * _DF), lambda i: (0, 0)),
        pl.BlockSpec((8 * _DF, 128), lambda i: (0, 0)),
    ],
    out_specs=pl.BlockSpec((10240 // 8, 128), lambda i: (0, 0)),
    out_shape=jax.ShapeDtypeStruct((10240 // 8, 128), jnp.float32),
)


def _tc1b_body(xw_ref, e0_ref, e1_ref, h_ref):
    dis = lax.rsqrt(e0_ref[...] + e1_ref[...] + 1.0)
    h_ref[...] = xw_ref[...] * dis


_NF = _NPAD // 8
_FBLK = _NF // 10
_DC8 = 8 * _DC


_tc1b = pl.pallas_call(
    _tc1b_body,
    grid=(2,),
    in_specs=[
        pl.BlockSpec((640, 128), lambda i: (i, 0)),
        pl.BlockSpec((None, 640, 128), lambda i: (0, i, 0)),
        pl.BlockSpec((None, 640, 128), lambda i: (1, i, 0)),
    ],
    out_specs=pl.BlockSpec((640, 128), lambda i: (i, 0)),
    out_shape=jax.ShapeDtypeStruct((10240 // 8, 128), jnp.float32),
)


def _tc2f_body(a0_ref, a1_ref, h_ref, e0_ref, e1_ref, b1_ref, z_ref):
    dis = lax.rsqrt(e0_ref[...] + e1_ref[...] + 1.0)
    agg = dis * (a0_ref[...] + a1_ref[...] - h_ref[...]) + b1_ref[...]
    z_ref[...] = dis * jnp.maximum(agg, 0.0)


_tc2f = pl.pallas_call(
    _tc2f_body,
    grid=(_NF // _FBLK,),
    in_specs=[
        pl.BlockSpec((None, _FBLK, 128), lambda i: (0, i, 0)),
        pl.BlockSpec((None, _FBLK, 128), lambda i: (1, i, 0)),
        pl.BlockSpec((_FBLK, 128), lambda i: (i, 0)),
        pl.BlockSpec((None, _FBLK, 128), lambda i: (0, i, 0)),
        pl.BlockSpec((None, _FBLK, 128), lambda i: (1, i, 0)),
        pl.BlockSpec((1, 128), lambda i: (0, 0)),
    ],
    out_specs=pl.BlockSpec((_FBLK, 128), lambda i: (i, 0)),
    out_shape=jax.ShapeDtypeStruct((_NF, 128), jnp.float32),
)


def _tc3f_body(a0_ref, a1_ref, z_ref, e0_ref, e1_ref, bd_ref, seg_ref,
               b2_ref, o_ref):
    dis = lax.rsqrt(e0_ref[...] + e1_ref[...] + 1.0)
    t = dis * (a0_ref[...] + a1_ref[...] - z_ref[...])
    logits = jnp.dot(t, bd_ref[...], preferred_element_type=jnp.float32)
    logits = logits + b2_ref[...]
    m = jnp.max(logits, axis=1, keepdims=True)
    e = jnp.exp(logits - m)
    s = jnp.dot(e, seg_ref[...], preferred_element_type=jnp.float32)
    o_ref[...] = logits - m - jnp.log(s)


_tc3f = pl.pallas_call(
    _tc3f_body,
    grid=(_NF // _FBLK,),
    in_specs=[
        pl.BlockSpec((None, _FBLK, 128), lambda i: (0, i, 0)),
        pl.BlockSpec((None, _FBLK, 128), lambda i: (1, i, 0)),
        pl.BlockSpec((_FBLK, 128), lambda i: (i, 0)),
        pl.BlockSpec((None, _FBLK, 128), lambda i: (0, i, 0)),
        pl.BlockSpec((None, _FBLK, 128), lambda i: (1, i, 0)),
        pl.BlockSpec((128, _DC8), lambda i: (0, 0)),
        pl.BlockSpec((_DC8, _DC8), lambda i: (0, 0)),
        pl.BlockSpec((1, _DC8), lambda i: (0, 0)),
    ],
    out_specs=pl.BlockSpec((_FBLK, _DC8), lambda i: (i, 0)),
    out_shape=jax.ShapeDtypeStruct((_NF, _DC8), jnp.float32),
)

_SEG = np.kron(np.eye(8, dtype=np.float32), np.ones((_DC, _DC), np.float32))


def kernel(x, edge_index, W1, b1, W2, b2):
    pad_e = _EPAD - _E
    pad_dst = _N + (jnp.arange(pad_e, dtype=jnp.int32) % (_NPAD - _N))
    pad_blk = jnp.stack([jnp.zeros((pad_e,), jnp.int32), pad_dst])
    ed = jnp.concatenate([edge_index.astype(jnp.int32), pad_blk], axis=1)
    ed = ed.reshape(2, _NW, _NCH, _CH)
    zeros1 = jnp.zeros((_NPAD,), jnp.float32)

    dege = _deg(ed, zeros1)
    degef = dege.reshape(_NC, _NF, 128)
    x_p = jnp.pad(x, ((0, _NPAD - _N), (0, 0))).reshape(_NF, 8 * _DF)
    w1k = jnp.kron(jnp.eye(8, dtype=W1.dtype), W1)
    xw = _tc1a(x_p, w1k)
    hf = _tc1b(xw, degef, degef)
    h = hf.reshape(_NPAD, _DH)
    a = _agg(h, ed)
    af = a.reshape(_NC, _NF, 128)
    zf = _tc2f(af, af, hf, degef, degef, jnp.tile(b1, 8).reshape(1, 128))
    a2 = _agg(zf.reshape(_NPAD, _DH), ed)
    a2f = a2.reshape(_NC, _NF, 128)
    bd = jnp.kron(jnp.eye(8, dtype=W2.dtype), W2)
    out = _tc3f(a2f, a2f, zf, degef, degef, bd,
                jnp.asarray(_SEG), jnp.tile(b2, 8).reshape(1, _DC8))
    return out.reshape(_NPAD, _DC)[:_N]
